# baseline (speedup 1.0000x reference)
import functools

import jax
import jax.numpy as jnp
from jax import lax
from jax.experimental import pallas as pl
from jax.experimental.pallas import tpu as pltpu
from jax.experimental.pallas import tpu_sc as plsc

N_NODES = 10000
N_EDGES = 320000
D = 128

NC = 2
NS = 16
NW = NC * NS
CH = 64
NCHT = N_EDGES // CH
NPAIRT = NCHT // 2
NPITER = 80
HALF_E = N_EDGES // 2
NPAIRH = NPAIRT // 2
NPITERH = 40
CHA = 128
NCHTA = N_EDGES // CHA
NPAIRA = NCHTA // 2
NPITERA = 40
N_PAD = 10240
STRIPE = N_PAD // NS

_MESH = plsc.VectorSubcoreMesh(core_axis_name="c", subcore_axis_name="s")


def _enc_body(x_ref, w_ref, b_ref, h1_ref, h21_ref, h22_ref):
    h = jnp.dot(x_ref[...], w_ref[...], preferred_element_type=jnp.float32)
    h = h + b_ref[...]
    h1_ref[...] = h[:, :D]
    h21_ref[...] = h[:, D:2 * D]
    h22_ref[...] = h[:, 2 * D:]


def _encode(x, w_enc, b_enc):
    rows = 2000
    grid = (N_NODES // rows,)
    out = jax.ShapeDtypeStruct((N_NODES, D), jnp.float32)
    return pl.pallas_call(
        _enc_body,
        grid=grid,
        in_specs=[
            pl.BlockSpec((rows, D), lambda i: (i, 0)),
            pl.BlockSpec((D, 3 * D), lambda i: (0, 0)),
            pl.BlockSpec((1, 3 * D), lambda i: (0, 0)),
        ],
        out_specs=[
            pl.BlockSpec((rows, D), lambda i: (i, 0)),
            pl.BlockSpec((rows, D), lambda i: (i, 0)),
            pl.BlockSpec((rows, D), lambda i: (i, 0)),
        ],
        out_shape=[out, out, out],
    )(x, w_enc, b_enc)


@functools.partial(
    pl.kernel,
    out_type=jax.ShapeDtypeStruct((N_EDGES, D), jnp.float32),
    mesh=_MESH,
    scratch_types=[
        pltpu.VMEM((2, CHA), jnp.int32),
        pltpu.VMEM((2, CHA), jnp.int32),
        pltpu.VMEM((CHA, D), jnp.float32),
        pltpu.VMEM((CHA, D), jnp.float32),
    ] + [pltpu.SemaphoreType.DMA] * 6,
)
def _sc1a(src_hbm, dst_hbm, h21_hbm, h22_hbm,
          s_hbm, idx_s, idx_d, srows_a, srows_b, si1, si2, s2, s4, s7, s8):
    c = lax.axis_index("c")
    sidx = lax.axis_index("s")
    wid = sidx * NC + c

    def body(j, carry):
        pid = j * NW + wid

        @pl.when(pid < NPAIRA)
        def _pair():
            i1 = pltpu.async_copy(src_hbm.at[pl.ds(2 * pid, 2)], idx_s, si1)
            i2 = pltpu.async_copy(dst_hbm.at[pl.ds(2 * pid, 2)], idx_d, si2)
            i1.wait()
            g2a = pltpu.async_copy(h21_hbm.at[idx_s.at[0]], srows_a, s2)
            g2b = pltpu.async_copy(h21_hbm.at[idx_s.at[1]], srows_b, s4)
            i2.wait()
            g2a.wait()
            g3a = pltpu.async_copy(h22_hbm.at[idx_d.at[0]], srows_a, s2,
                                   add=True)
            g2b.wait()
            g3b = pltpu.async_copy(h22_hbm.at[idx_d.at[1]], srows_b, s4,
                                   add=True)
            g3a.wait()
            wa = pltpu.async_copy(srows_a, s_hbm.at[pl.ds(2 * pid * CHA, CHA)],
                                  s7)
            g3b.wait()
            wb = pltpu.async_copy(srows_b,
                                  s_hbm.at[pl.ds((2 * pid + 1) * CHA, CHA)],
                                  s8)
            wa.wait()
            wb.wait()

        return carry

    lax.fori_loop(0, NPITERA, body, 0)


@functools.partial(
    pl.kernel,
    out_type=jax.ShapeDtypeStruct((NC * N_PAD, D), jnp.float32),
    mesh=_MESH,
    scratch_types=[
        pltpu.VMEM((2, CHA), jnp.int32),
        pltpu.VMEM((2, CHA), jnp.int32),
        pltpu.VMEM((CHA, D), jnp.float32),
        pltpu.VMEM((CHA, D), jnp.float32),
        pltpu.VMEM_SHARED((N_PAD, D), jnp.float32),
    ] + [pltpu.SemaphoreType.DMA] * 6,
)
def _sc1b(src_hbm, dst_hbm, h1_hbm, z_hbm,
          agg_hbm, idx_s, idx_d, rows_a, rows_b, aggsh,
          si1, si2, s1, s3, s5, s6):
    c = lax.axis_index("c")
    sidx = lax.axis_index("s")
    wid = sidx * NC + c
    tid = sidx

    pltpu.sync_copy(z_hbm, aggsh.at[pl.ds(tid * STRIPE, STRIPE)])
    plsc.subcore_barrier()

    def body(j, carry):
        pid = j * NW + wid

        @pl.when(pid < NPAIRA)
        def _pair():
            i1 = pltpu.async_copy(src_hbm.at[pl.ds(2 * pid, 2)], idx_s, si1)
            i2 = pltpu.async_copy(dst_hbm.at[pl.ds(2 * pid, 2)], idx_d, si2)
            i1.wait()
            g1a = pltpu.async_copy(h1_hbm.at[idx_s.at[0]], rows_a, s1)
            g1b = pltpu.async_copy(h1_hbm.at[idx_s.at[1]], rows_b, s3)
            i2.wait()
            g1a.wait()
            sca = pltpu.async_copy(rows_a, aggsh.at[idx_d.at[0]], s5,
                                   add=True)
            g1b.wait()
            scb = pltpu.async_copy(rows_b, aggsh.at[idx_d.at[1]], s6,
                                   add=True)
            sca.wait()
            scb.wait()

        return carry

    lax.fori_loop(0, NPITERA, body, 0)

    plsc.subcore_barrier()
    pltpu.sync_copy(aggsh.at[pl.ds(tid * STRIPE, STRIPE)],
                    agg_hbm.at[pl.ds(c * N_PAD + tid * STRIPE, STRIPE)])


def _umlp_body(s_ref, w0, b0, w1, b1, w2, b2, u_ref):
    h = jnp.tanh(jnp.dot(s_ref[...], w0[...],
                         preferred_element_type=jnp.float32) + b0[...])
    h = jnp.maximum(jnp.dot(h, w1[...],
                            preferred_element_type=jnp.float32) + b1[...], 0.0)
    u_ref[...] = jnp.dot(h, w2[...],
                         preferred_element_type=jnp.float32) + b2[...]


def _umlp(s, half, w0, b0, w1, b1, w2, b2):
    rows = 2000
    grid = (HALF_E // rows,)
    off = half * (HALF_E // rows)
    wspec = pl.BlockSpec((D, D), lambda i: (0, 0))
    bspec = pl.BlockSpec((1, D), lambda i: (0, 0))
    return pl.pallas_call(
        _umlp_body,
        grid=grid,
        in_specs=[pl.BlockSpec((rows, D), lambda i: (i + off, 0)),
                  wspec, bspec, wspec, bspec, wspec, bspec],
        out_specs=pl.BlockSpec((rows, D), lambda i: (i, 0)),
        out_shape=jax.ShapeDtypeStruct((HALF_E, D), jnp.float32),
    )(s, w0, b0, w1, b1, w2, b2)


@functools.partial(
    pl.kernel,
    out_type=jax.ShapeDtypeStruct((NC * N_PAD, D), jnp.float32),
    mesh=_MESH,
    scratch_types=[
        pltpu.VMEM((2, CH), jnp.int32),
        pltpu.VMEM((2, CH), jnp.int32),
        pltpu.VMEM((2 * CH, D), jnp.float32),
        pltpu.VMEM((2 * CH, D), jnp.float32),
        pltpu.VMEM_SHARED((N_PAD, D), jnp.float32),
    ] + [pltpu.SemaphoreType.DMA] * 6,
)
def _sc2(src_hbm, u_hbm, z_hbm, usum_hbm, idx_a, idx_b, rows_a, rows_b,
         ussh, si1, si2, sl1, sl2, sca, scb):
    c = lax.axis_index("c")
    sidx = lax.axis_index("s")
    wid = sidx * NC + c
    tid = sidx

    pltpu.sync_copy(z_hbm, ussh.at[pl.ds(tid * STRIPE, STRIPE)])
    plsc.subcore_barrier()

    def body(m, carry):
        pid_a = (2 * m) * NW + wid
        pid_b = (2 * m + 1) * NW + wid

        @pl.when(pid_a < NPAIRH)
        def _a():
            i_a = pltpu.async_copy(src_hbm.at[pl.ds(2 * pid_a, 2)], idx_a,
                                   si1)
            l_a = pltpu.async_copy(u_hbm.at[pl.ds(2 * pid_a * CH, 2 * CH)],
                                   rows_a, sl1)

            @pl.when(pid_b < NPAIRH)
            def _b():
                i_b = pltpu.async_copy(src_hbm.at[pl.ds(2 * pid_b, 2)],
                                       idx_b, si2)
                l_b = pltpu.async_copy(u_hbm.at[pl.ds(2 * pid_b * CH, 2 * CH)],
                                       rows_b, sl2)
                i_a.wait()
                l_a.wait()
                sa1 = pltpu.async_copy(rows_a.at[pl.ds(0, CH)],
                                       ussh.at[idx_a.at[0]], sca, add=True)
                sa2 = pltpu.async_copy(rows_a.at[pl.ds(CH, CH)],
                                       ussh.at[idx_a.at[1]], sca, add=True)
                i_b.wait()
                l_b.wait()
                sb1 = pltpu.async_copy(rows_b.at[pl.ds(0, CH)],
                                       ussh.at[idx_b.at[0]], scb, add=True)
                sb2 = pltpu.async_copy(rows_b.at[pl.ds(CH, CH)],
                                       ussh.at[idx_b.at[1]], scb, add=True)
                sa1.wait()
                sa2.wait()
                sb1.wait()
                sb2.wait()

            @pl.when(jnp.logical_not(pid_b < NPAIRH))
            def _a_only():
                i_a.wait()
                l_a.wait()
                sa1 = pltpu.async_copy(rows_a.at[pl.ds(0, CH)],
                                       ussh.at[idx_a.at[0]], sca, add=True)
                sa2 = pltpu.async_copy(rows_a.at[pl.ds(CH, CH)],
                                       ussh.at[idx_a.at[1]], sca, add=True)
                sa1.wait()
                sa2.wait()

        return carry

    lax.fori_loop(0, NPITERH, body, 0)

    plsc.subcore_barrier()
    pltpu.sync_copy(ussh.at[pl.ds(tid * STRIPE, STRIPE)],
                    usum_hbm.at[pl.ds(c * N_PAD + tid * STRIPE, STRIPE)])


def _kdot_body(agg_ref, usum_ref, w0, b0, w1, b1, w2, b2, out_ref):
    a = agg_ref[0] + agg_ref[1]
    us = usum_ref[0] + usum_ref[1]
    h = jnp.tanh(jnp.dot(a, w0[...],
                         preferred_element_type=jnp.float32) + b0[...])
    h = jnp.maximum(jnp.dot(h, w1[...],
                            preferred_element_type=jnp.float32) + b1[...], 0.0)
    k = jnp.dot(h, w2[...], preferred_element_type=jnp.float32) + b2[...]
    part = jnp.sum(k * us).reshape(1, 1)

    @pl.when(pl.program_id(0) == 0)
    def _():
        out_ref[...] = jnp.zeros((1, 1), jnp.float32)

    out_ref[...] += part


def _kdot(agg, usum, w0, b0, w1, b1, w2, b2):
    rows = 2048
    grid = (N_PAD // rows,)
    wspec = pl.BlockSpec((D, D), lambda i: (0, 0))
    bspec = pl.BlockSpec((1, D), lambda i: (0, 0))
    out = pl.pallas_call(
        _kdot_body,
        grid=grid,
        in_specs=[pl.BlockSpec((NC, rows, D), lambda i: (0, i, 0)),
                  pl.BlockSpec((NC, rows, D), lambda i: (0, i, 0)),
                  wspec, bspec, wspec, bspec, wspec, bspec],
        out_specs=pl.BlockSpec((1, 1), lambda i: (0, 0)),
        out_shape=jax.ShapeDtypeStruct((1, 1), jnp.float32),
    )(agg, usum, w0, b0, w1, b1, w2, b2)
    return out[0, 0]


def kernel(x, edge_index, e,
           Wk0, bk0, Wk1, bk1, Wk2, bk2,
           Wu0, bu0, Wu1, bu1, Wu2, bu2,
           WencK, bencK, WencP1, bencP1, WencP2, bencP2):
    src = edge_index[0].reshape(NCHT, CH)
    dst = edge_index[1].reshape(NCHT, CH)
    src_a = edge_index[0].reshape(NCHTA, CHA)
    dst_a = edge_index[1].reshape(NCHTA, CHA)

    w_enc = jnp.concatenate([WencK.T, WencP1.T, WencP2.T], axis=1)
    b_enc = jnp.concatenate([bencK, bencP1, bencP2])[None, :]
    h1, h21, h22 = _encode(x, w_enc, b_enc)

    z = jnp.zeros((STRIPE, D), jnp.float32)
    s = _sc1a(src_a, dst_a, h21, h22)
    agg = _sc1b(src_a, dst_a, h1, z)

    uw = (Wu0.T, bu0[None, :], Wu1.T, bu1[None, :], Wu2.T, bu2[None, :])
    u0 = _umlp(s, 0, *uw)
    u1 = _umlp(s, 1, *uw)
    usum0 = _sc2(src[:NCHT // 2], u0, z)
    usum1 = _sc2(src[NCHT // 2:], u1, z)

    agg3 = agg.reshape(NC, N_PAD, D)
    usum3 = (usum0 + usum1).reshape(NC, N_PAD, D)
    return _kdot(agg3, usum3,
                 Wk0.T, bk0[None, :], Wk1.T, bk1[None, :], Wk2.T, bk2[None, :])


# speedup vs baseline: 1.1785x; 1.1785x over previous
import functools

import jax
import jax.numpy as jnp
from jax import lax
from jax.experimental import pallas as pl
from jax.experimental.pallas import tpu as pltpu
from jax.experimental.pallas import tpu_sc as plsc

N_NODES = 10000
N_EDGES = 320000
D = 128

NC = 2
NS = 16
NW = NC * NS
CH = 64
NCHT = N_EDGES // CH
NPAIRT = NCHT // 2
NPITER = 80
HALF_E = N_EDGES // 2
NPAIRH = NPAIRT // 2
NPITERH = 40
CHA = 128
NCHTA = N_EDGES // CHA
NPAIRA = NCHTA // 2
NPITERA = 40
N_PAD = 10240
STRIPE = N_PAD // NS

_MESH = plsc.VectorSubcoreMesh(core_axis_name="c", subcore_axis_name="s")


def _enc_body(x_ref, w_ref, b_ref, h1_ref, h21_ref, h22_ref):
    h = jnp.dot(x_ref[...], w_ref[...], preferred_element_type=jnp.float32)
    h = h + b_ref[...]
    h1_ref[...] = h[:, :D]
    h21_ref[...] = h[:, D:2 * D]
    h22_ref[...] = h[:, 2 * D:]


def _encode(x, w_enc, b_enc):
    rows = 2000
    grid = (N_NODES // rows,)
    return pl.pallas_call(
        _enc_body,
        grid=grid,
        in_specs=[
            pl.BlockSpec((rows, D), lambda i: (i, 0)),
            pl.BlockSpec((D, 3 * D), lambda i: (0, 0)),
            pl.BlockSpec((1, 3 * D), lambda i: (0, 0)),
        ],
        out_specs=[
            pl.BlockSpec((rows, D), lambda i: (i, 0)),
            pl.BlockSpec((rows, D), lambda i: (i, 0)),
            pl.BlockSpec((rows, D), lambda i: (i, 0)),
        ],
        out_shape=[jax.ShapeDtypeStruct((N_NODES, D), jnp.float32)] * 3,
    )(x, w_enc, b_enc)


@functools.partial(
    pl.kernel,
    out_type=jax.ShapeDtypeStruct((N_EDGES, D), jnp.float32),
    mesh=_MESH,
    scratch_types=[
        pltpu.VMEM((2, CHA), jnp.int32),
        pltpu.VMEM((2, CHA), jnp.int32),
        pltpu.VMEM((CHA, D), jnp.float32),
        pltpu.VMEM((CHA, D), jnp.float32),
    ] + [pltpu.SemaphoreType.DMA] * 6,
)
def _sc1a(src_hbm, dst_hbm, h21_hbm, h22_hbm,
          s_hbm, idx_s, idx_d, srows_a, srows_b, si1, si2, s2, s4, s7, s8):
    c = lax.axis_index("c")
    sidx = lax.axis_index("s")
    wid = sidx * NC + c

    def body(j, carry):
        pid = j * NW + wid

        @pl.when(pid < NPAIRA)
        def _pair():
            i1 = pltpu.async_copy(src_hbm.at[pl.ds(2 * pid, 2)], idx_s, si1)
            i2 = pltpu.async_copy(dst_hbm.at[pl.ds(2 * pid, 2)], idx_d, si2)
            i1.wait()
            g2a = pltpu.async_copy(h21_hbm.at[idx_s.at[0]], srows_a, s2)
            g2b = pltpu.async_copy(h21_hbm.at[idx_s.at[1]], srows_b, s4)
            i2.wait()
            g2a.wait()
            g3a = pltpu.async_copy(h22_hbm.at[idx_d.at[0]], srows_a, s2,
                                   add=True)
            g2b.wait()
            g3b = pltpu.async_copy(h22_hbm.at[idx_d.at[1]], srows_b, s4,
                                   add=True)
            g3a.wait()
            wa = pltpu.async_copy(srows_a, s_hbm.at[pl.ds(2 * pid * CHA, CHA)],
                                  s7)
            g3b.wait()
            wb = pltpu.async_copy(srows_b,
                                  s_hbm.at[pl.ds((2 * pid + 1) * CHA, CHA)],
                                  s8)
            wa.wait()
            wb.wait()

        return carry

    lax.fori_loop(0, NPITERA, body, 0)


@functools.partial(
    pl.kernel,
    out_type=jax.ShapeDtypeStruct((NC * N_PAD, D), jnp.float32),
    mesh=_MESH,
    scratch_types=[
        pltpu.VMEM((2, CHA), jnp.int32),
        pltpu.VMEM((2, CHA), jnp.int32),
        pltpu.VMEM((CHA, D), jnp.float32),
        pltpu.VMEM((CHA, D), jnp.float32),
        pltpu.VMEM_SHARED((N_PAD, D), jnp.float32),
    ] + [pltpu.SemaphoreType.DMA] * 6,
)
def _sc1b(src_hbm, dst_hbm, h1_hbm, z_hbm,
          agg_hbm, idx_s, idx_d, rows_a, rows_b, aggsh,
          si1, si2, s1, s3, s5, s6):
    c = lax.axis_index("c")
    sidx = lax.axis_index("s")
    wid = sidx * NC + c
    tid = sidx

    pltpu.sync_copy(z_hbm, aggsh.at[pl.ds(tid * STRIPE, STRIPE)])
    plsc.subcore_barrier()

    def body(j, carry):
        pid = j * NW + wid

        @pl.when(pid < NPAIRA)
        def _pair():
            i1 = pltpu.async_copy(src_hbm.at[pl.ds(2 * pid, 2)], idx_s, si1)
            i2 = pltpu.async_copy(dst_hbm.at[pl.ds(2 * pid, 2)], idx_d, si2)
            i1.wait()
            g1a = pltpu.async_copy(h1_hbm.at[idx_s.at[0]], rows_a, s1)
            g1b = pltpu.async_copy(h1_hbm.at[idx_s.at[1]], rows_b, s3)
            i2.wait()
            g1a.wait()
            sca = pltpu.async_copy(rows_a, aggsh.at[idx_d.at[0]], s5,
                                   add=True)
            g1b.wait()
            scb = pltpu.async_copy(rows_b, aggsh.at[idx_d.at[1]], s6,
                                   add=True)
            sca.wait()
            scb.wait()

        return carry

    lax.fori_loop(0, NPITERA, body, 0)

    plsc.subcore_barrier()
    pltpu.sync_copy(aggsh.at[pl.ds(tid * STRIPE, STRIPE)],
                    agg_hbm.at[pl.ds(c * N_PAD + tid * STRIPE, STRIPE)])


def _umlp_body(s_ref, w0, b0, w1, b1, w2, b2, u_ref):
    h = jnp.tanh(jnp.dot(s_ref[...], w0[...],
                         preferred_element_type=jnp.float32) + b0[...])
    h = jnp.maximum(jnp.dot(h, w1[...],
                            preferred_element_type=jnp.float32) + b1[...], 0.0)
    u_ref[...] = jnp.dot(h, w2[...],
                         preferred_element_type=jnp.float32) + b2[...]


def _umlp(s, w0, b0, w1, b1, w2, b2):
    rows = 2000
    grid = (N_EDGES // rows,)
    wspec = pl.BlockSpec((D, D), lambda i: (0, 0))
    bspec = pl.BlockSpec((1, D), lambda i: (0, 0))
    return pl.pallas_call(
        _umlp_body,
        grid=grid,
        in_specs=[pl.BlockSpec((rows, D), lambda i: (i, 0)),
                  wspec, bspec, wspec, bspec, wspec, bspec],
        out_specs=pl.BlockSpec((rows, D), lambda i: (i, 0)),
        out_shape=jax.ShapeDtypeStruct((N_EDGES, D), jnp.float32),
    )(s, w0, b0, w1, b1, w2, b2)


@functools.partial(
    pl.kernel,
    out_type=jax.ShapeDtypeStruct((NC * N_PAD, D), jnp.float32),
    mesh=_MESH,
    scratch_types=[
        pltpu.VMEM((2, CH), jnp.int32),
        pltpu.VMEM((2, CH), jnp.int32),
        pltpu.VMEM((2 * CH, D), jnp.float32),
        pltpu.VMEM((2 * CH, D), jnp.float32),
        pltpu.VMEM_SHARED((N_PAD, D), jnp.float32),
    ] + [pltpu.SemaphoreType.DMA] * 6,
)
def _sc2(src_hbm, u_hbm, z_hbm, usum_hbm, idx_a, idx_b, rows_a, rows_b,
         ussh, si1, si2, sl1, sl2, sca, scb):
    c = lax.axis_index("c")
    sidx = lax.axis_index("s")
    wid = sidx * NC + c
    tid = sidx

    pltpu.sync_copy(z_hbm, ussh.at[pl.ds(tid * STRIPE, STRIPE)])
    plsc.subcore_barrier()

    def body(m, carry):
        pid_a = (2 * m) * NW + wid
        pid_b = (2 * m + 1) * NW + wid

        @pl.when(pid_a < NPAIRT)
        def _a():
            i_a = pltpu.async_copy(src_hbm.at[pl.ds(2 * pid_a, 2)], idx_a,
                                   si1)
            l_a = pltpu.async_copy(u_hbm.at[pl.ds(2 * pid_a * CH, 2 * CH)],
                                   rows_a, sl1)

            @pl.when(pid_b < NPAIRT)
            def _b():
                i_b = pltpu.async_copy(src_hbm.at[pl.ds(2 * pid_b, 2)],
                                       idx_b, si2)
                l_b = pltpu.async_copy(u_hbm.at[pl.ds(2 * pid_b * CH, 2 * CH)],
                                       rows_b, sl2)
                i_a.wait()
                l_a.wait()
                sa1 = pltpu.async_copy(rows_a.at[pl.ds(0, CH)],
                                       ussh.at[idx_a.at[0]], sca, add=True)
                sa2 = pltpu.async_copy(rows_a.at[pl.ds(CH, CH)],
                                       ussh.at[idx_a.at[1]], sca, add=True)
                i_b.wait()
                l_b.wait()
                sb1 = pltpu.async_copy(rows_b.at[pl.ds(0, CH)],
                                       ussh.at[idx_b.at[0]], scb, add=True)
                sb2 = pltpu.async_copy(rows_b.at[pl.ds(CH, CH)],
                                       ussh.at[idx_b.at[1]], scb, add=True)
                sa1.wait()
                sa2.wait()
                sb1.wait()
                sb2.wait()

            @pl.when(jnp.logical_not(pid_b < NPAIRT))
            def _a_only():
                i_a.wait()
                l_a.wait()
                sa1 = pltpu.async_copy(rows_a.at[pl.ds(0, CH)],
                                       ussh.at[idx_a.at[0]], sca, add=True)
                sa2 = pltpu.async_copy(rows_a.at[pl.ds(CH, CH)],
                                       ussh.at[idx_a.at[1]], sca, add=True)
                sa1.wait()
                sa2.wait()

        return carry

    lax.fori_loop(0, NPITER // 2, body, 0)

    plsc.subcore_barrier()
    pltpu.sync_copy(ussh.at[pl.ds(tid * STRIPE, STRIPE)],
                    usum_hbm.at[pl.ds(c * N_PAD + tid * STRIPE, STRIPE)])


def _kdot_body(agg_ref, usum_ref, w0, b0, w1, b1, w2, b2, out_ref):
    a = agg_ref[0] + agg_ref[1]
    us = usum_ref[0] + usum_ref[1]
    h = jnp.tanh(jnp.dot(a, w0[...],
                         preferred_element_type=jnp.float32) + b0[...])
    h = jnp.maximum(jnp.dot(h, w1[...],
                            preferred_element_type=jnp.float32) + b1[...], 0.0)
    k = jnp.dot(h, w2[...], preferred_element_type=jnp.float32) + b2[...]
    part = jnp.sum(k * us).reshape(1, 1)

    @pl.when(pl.program_id(0) == 0)
    def _():
        out_ref[...] = jnp.zeros((1, 1), jnp.float32)

    out_ref[...] += part


def _kdot(agg, usum, w0, b0, w1, b1, w2, b2):
    rows = 2048
    grid = (N_PAD // rows,)
    wspec = pl.BlockSpec((D, D), lambda i: (0, 0))
    bspec = pl.BlockSpec((1, D), lambda i: (0, 0))
    out = pl.pallas_call(
        _kdot_body,
        grid=grid,
        in_specs=[pl.BlockSpec((NC, rows, D), lambda i: (0, i, 0)),
                  pl.BlockSpec((NC, rows, D), lambda i: (0, i, 0)),
                  wspec, bspec, wspec, bspec, wspec, bspec],
        out_specs=pl.BlockSpec((1, 1), lambda i: (0, 0)),
        out_shape=jax.ShapeDtypeStruct((1, 1), jnp.float32),
    )(agg, usum, w0, b0, w1, b1, w2, b2)
    return out[0, 0]


def kernel(x, edge_index, e,
           Wk0, bk0, Wk1, bk1, Wk2, bk2,
           Wu0, bu0, Wu1, bu1, Wu2, bu2,
           WencK, bencK, WencP1, bencP1, WencP2, bencP2):
    src = edge_index[0].reshape(NCHT, CH)
    dst = edge_index[1].reshape(NCHT, CH)
    src_a = edge_index[0].reshape(NCHTA, CHA)
    dst_a = edge_index[1].reshape(NCHTA, CHA)

    w_enc = jnp.concatenate([WencK.T, WencP1.T, WencP2.T], axis=1)
    b_enc = jnp.concatenate([bencK, bencP1, bencP2])[None, :]
    h1, h21, h22 = _encode(x, w_enc, b_enc)

    z = jnp.zeros((STRIPE, D), jnp.float32)
    s = _sc1a(src_a, dst_a, h21, h22)
    agg = _sc1b(src_a, dst_a, h1, z)

    u = _umlp(s, Wu0.T, bu0[None, :], Wu1.T, bu1[None, :], Wu2.T, bu2[None, :])

    usum = _sc2(src, u, z)

    agg3 = agg.reshape(NC, N_PAD, D)
    usum3 = usum.reshape(NC, N_PAD, D)
    return _kdot(agg3, usum3,
                 Wk0.T, bk0[None, :], Wk1.T, bk1[None, :], Wk2.T, bk2[None, :])


# speedup vs baseline: 1.2484x; 1.0593x over previous
import functools

import jax
import jax.numpy as jnp
from jax import lax
from jax.experimental import pallas as pl
from jax.experimental.pallas import tpu as pltpu
from jax.experimental.pallas import tpu_sc as plsc

N_NODES = 10000
N_EDGES = 320000
D = 128

NC = 2
NS = 16
NW = NC * NS
CH = 64
NCHT = N_EDGES // CH
NPAIRT = NCHT // 2
NPITER = 80
HALF_E = N_EDGES // 2
NPAIRH = NPAIRT // 2
NPITERH = 40
CHA = 128
NCHTA = N_EDGES // CHA
NPAIRA = NCHTA // 2
NPITERA = 40
N_PAD = 10240
STRIPE = N_PAD // NS

_MESH = plsc.VectorSubcoreMesh(core_axis_name="c", subcore_axis_name="s")


def _enc_body(x_ref, w_ref, b_ref, h1_ref, h21_ref, h22_ref):
    h = jnp.dot(x_ref[...], w_ref[...], preferred_element_type=jnp.float32)
    h = h + b_ref[...]
    h1_ref[...] = h[:, :D]
    h21_ref[...] = h[:, D:2 * D]
    h22_ref[...] = h[:, 2 * D:]


def _encode(x, w_enc, b_enc):
    rows = 2000
    grid = (N_NODES // rows,)
    return pl.pallas_call(
        _enc_body,
        grid=grid,
        in_specs=[
            pl.BlockSpec((rows, D), lambda i: (i, 0)),
            pl.BlockSpec((D, 3 * D), lambda i: (0, 0)),
            pl.BlockSpec((1, 3 * D), lambda i: (0, 0)),
        ],
        out_specs=[
            pl.BlockSpec((rows, D), lambda i: (i, 0)),
            pl.BlockSpec((rows, D), lambda i: (i, 0)),
            pl.BlockSpec((rows, D), lambda i: (i, 0)),
        ],
        out_shape=[jax.ShapeDtypeStruct((N_NODES, D), jnp.float32)] * 3,
    )(x, w_enc, b_enc)


@functools.partial(
    pl.kernel,
    out_type=jax.ShapeDtypeStruct((N_EDGES, D), jnp.float32),
    mesh=_MESH,
    scratch_types=[
        pltpu.VMEM((2, CHA), jnp.int32),
        pltpu.VMEM((2, CHA), jnp.int32),
        pltpu.VMEM((2, CHA), jnp.int32),
        pltpu.VMEM((2, CHA), jnp.int32),
        pltpu.VMEM((CHA, D), jnp.float32),
        pltpu.VMEM((CHA, D), jnp.float32),
        pltpu.VMEM((CHA, D), jnp.float32),
        pltpu.VMEM((CHA, D), jnp.float32),
    ] + [pltpu.SemaphoreType.DMA] * 12,
)
def _sc1a(src_hbm, dst_hbm, h21_hbm, h22_hbm, s_hbm,
          ixs0, ixd0, ixs1, ixd1, ra, rb, rc, rd,
          i0, i1, i2, i3, ga, gb, gc, gd, wsa, wsb, wsc, wsd):
    c = lax.axis_index("c")
    sidx = lax.axis_index("s")
    wid = sidx * NC + c

    def body(m, carry):
        p0 = (2 * m) * NW + wid
        p1 = (2 * m + 1) * NW + wid

        @pl.when(p0 < NPAIRA)
        def _p0():
            ia = pltpu.async_copy(src_hbm.at[pl.ds(2 * p0, 2)], ixs0, i0)
            ib = pltpu.async_copy(dst_hbm.at[pl.ds(2 * p0, 2)], ixd0, i1)

            @pl.when(p1 < NPAIRA)
            def _pre1():
                pltpu.async_copy(src_hbm.at[pl.ds(2 * p1, 2)], ixs1, i2)
                pltpu.async_copy(dst_hbm.at[pl.ds(2 * p1, 2)], ixd1, i3)

            ia.wait()
            g_a = pltpu.async_copy(h21_hbm.at[ixs0.at[0]], ra, ga)
            g_b = pltpu.async_copy(h21_hbm.at[ixs0.at[1]], rb, gb)

            @pl.when(p1 < NPAIRA)
            def _g1():
                pltpu.make_async_copy(src_hbm.at[pl.ds(2 * p1, 2)], ixs1,
                                      i2).wait()
                pltpu.async_copy(h21_hbm.at[ixs1.at[0]], rc, gc)
                pltpu.async_copy(h21_hbm.at[ixs1.at[1]], rd, gd)

            ib.wait()
            g_a.wait()
            a3 = pltpu.async_copy(h22_hbm.at[ixd0.at[0]], ra, ga, add=True)
            g_b.wait()
            b3 = pltpu.async_copy(h22_hbm.at[ixd0.at[1]], rb, gb, add=True)

            @pl.when(p1 < NPAIRA)
            def _g2():
                pltpu.make_async_copy(dst_hbm.at[pl.ds(2 * p1, 2)], ixd1,
                                      i3).wait()
                pltpu.make_async_copy(h21_hbm.at[ixs1.at[0]], rc, gc).wait()
                pltpu.async_copy(h22_hbm.at[ixd1.at[0]], rc, gc, add=True)
                pltpu.make_async_copy(h21_hbm.at[ixs1.at[1]], rd, gd).wait()
                pltpu.async_copy(h22_hbm.at[ixd1.at[1]], rd, gd, add=True)

            a3.wait()
            w_a = pltpu.async_copy(ra, s_hbm.at[pl.ds(2 * p0 * CHA, CHA)],
                                   wsa)
            b3.wait()
            w_b = pltpu.async_copy(rb,
                                   s_hbm.at[pl.ds((2 * p0 + 1) * CHA, CHA)],
                                   wsb)

            @pl.when(p1 < NPAIRA)
            def _w1():
                pltpu.make_async_copy(h22_hbm.at[ixd1.at[0]], rc, gc).wait()
                pltpu.async_copy(rc, s_hbm.at[pl.ds(2 * p1 * CHA, CHA)], wsc)
                pltpu.make_async_copy(h22_hbm.at[ixd1.at[1]], rd, gd).wait()
                pltpu.async_copy(rd,
                                 s_hbm.at[pl.ds((2 * p1 + 1) * CHA, CHA)],
                                 wsd)

            w_a.wait()
            w_b.wait()

            @pl.when(p1 < NPAIRA)
            def _dr1():
                pltpu.make_async_copy(
                    rc, s_hbm.at[pl.ds(2 * p1 * CHA, CHA)], wsc).wait()
                pltpu.make_async_copy(
                    rd, s_hbm.at[pl.ds((2 * p1 + 1) * CHA, CHA)], wsd).wait()

        return carry

    lax.fori_loop(0, NPITERA // 2, body, 0)


@functools.partial(
    pl.kernel,
    out_type=jax.ShapeDtypeStruct((NC * N_PAD, D), jnp.float32),
    mesh=_MESH,
    scratch_types=[
        pltpu.VMEM((2, CHA), jnp.int32),
        pltpu.VMEM((2, CHA), jnp.int32),
        pltpu.VMEM((CHA, D), jnp.float32),
        pltpu.VMEM((CHA, D), jnp.float32),
        pltpu.VMEM_SHARED((N_PAD, D), jnp.float32),
    ] + [pltpu.SemaphoreType.DMA] * 6,
)
def _sc1b(src_hbm, dst_hbm, h1_hbm, z_hbm,
          agg_hbm, idx_s, idx_d, rows_a, rows_b, aggsh,
          si1, si2, s1, s3, s5, s6):
    c = lax.axis_index("c")
    sidx = lax.axis_index("s")
    wid = sidx * NC + c
    tid = sidx

    pltpu.sync_copy(z_hbm, aggsh.at[pl.ds(tid * STRIPE, STRIPE)])
    plsc.subcore_barrier()

    def body(j, carry):
        pid = j * NW + wid

        @pl.when(pid < NPAIRA)
        def _pair():
            i1 = pltpu.async_copy(src_hbm.at[pl.ds(2 * pid, 2)], idx_s, si1)
            i2 = pltpu.async_copy(dst_hbm.at[pl.ds(2 * pid, 2)], idx_d, si2)
            i1.wait()
            g1a = pltpu.async_copy(h1_hbm.at[idx_s.at[0]], rows_a, s1)
            g1b = pltpu.async_copy(h1_hbm.at[idx_s.at[1]], rows_b, s3)
            i2.wait()
            g1a.wait()
            sca = pltpu.async_copy(rows_a, aggsh.at[idx_d.at[0]], s5,
                                   add=True)
            g1b.wait()
            scb = pltpu.async_copy(rows_b, aggsh.at[idx_d.at[1]], s6,
                                   add=True)
            sca.wait()
            scb.wait()

        return carry

    lax.fori_loop(0, NPITERA, body, 0)

    plsc.subcore_barrier()
    pltpu.sync_copy(aggsh.at[pl.ds(tid * STRIPE, STRIPE)],
                    agg_hbm.at[pl.ds(c * N_PAD + tid * STRIPE, STRIPE)])


def _umlp_body(s_ref, w0, b0, w1, b1, w2, b2, u_ref):
    h = jnp.tanh(jnp.dot(s_ref[...], w0[...],
                         preferred_element_type=jnp.float32) + b0[...])
    h = jnp.maximum(jnp.dot(h, w1[...],
                            preferred_element_type=jnp.float32) + b1[...], 0.0)
    u_ref[...] = jnp.dot(h, w2[...],
                         preferred_element_type=jnp.float32) + b2[...]


def _umlp(s, w0, b0, w1, b1, w2, b2):
    rows = 2000
    grid = (N_EDGES // rows,)
    wspec = pl.BlockSpec((D, D), lambda i: (0, 0))
    bspec = pl.BlockSpec((1, D), lambda i: (0, 0))
    return pl.pallas_call(
        _umlp_body,
        grid=grid,
        in_specs=[pl.BlockSpec((rows, D), lambda i: (i, 0)),
                  wspec, bspec, wspec, bspec, wspec, bspec],
        out_specs=pl.BlockSpec((rows, D), lambda i: (i, 0)),
        out_shape=jax.ShapeDtypeStruct((N_EDGES, D), jnp.float32),
    )(s, w0, b0, w1, b1, w2, b2)


@functools.partial(
    pl.kernel,
    out_type=jax.ShapeDtypeStruct((NC * N_PAD, D), jnp.float32),
    mesh=_MESH,
    scratch_types=[
        pltpu.VMEM((2, CH), jnp.int32),
        pltpu.VMEM((2, CH), jnp.int32),
        pltpu.VMEM((2 * CH, D), jnp.float32),
        pltpu.VMEM((2 * CH, D), jnp.float32),
        pltpu.VMEM_SHARED((N_PAD, D), jnp.float32),
    ] + [pltpu.SemaphoreType.DMA] * 6,
)
def _sc2(src_hbm, u_hbm, z_hbm, usum_hbm, idx_a, idx_b, rows_a, rows_b,
         ussh, si1, si2, sl1, sl2, sca, scb):
    c = lax.axis_index("c")
    sidx = lax.axis_index("s")
    wid = sidx * NC + c
    tid = sidx

    pltpu.sync_copy(z_hbm, ussh.at[pl.ds(tid * STRIPE, STRIPE)])
    plsc.subcore_barrier()

    def body(m, carry):
        pid_a = (2 * m) * NW + wid
        pid_b = (2 * m + 1) * NW + wid

        @pl.when(pid_a < NPAIRT)
        def _a():
            i_a = pltpu.async_copy(src_hbm.at[pl.ds(2 * pid_a, 2)], idx_a,
                                   si1)
            l_a = pltpu.async_copy(u_hbm.at[pl.ds(2 * pid_a * CH, 2 * CH)],
                                   rows_a, sl1)

            @pl.when(pid_b < NPAIRT)
            def _b():
                i_b = pltpu.async_copy(src_hbm.at[pl.ds(2 * pid_b, 2)],
                                       idx_b, si2)
                l_b = pltpu.async_copy(u_hbm.at[pl.ds(2 * pid_b * CH, 2 * CH)],
                                       rows_b, sl2)
                i_a.wait()
                l_a.wait()
                sa1 = pltpu.async_copy(rows_a.at[pl.ds(0, CH)],
                                       ussh.at[idx_a.at[0]], sca, add=True)
                sa2 = pltpu.async_copy(rows_a.at[pl.ds(CH, CH)],
                                       ussh.at[idx_a.at[1]], sca, add=True)
                i_b.wait()
                l_b.wait()
                sb1 = pltpu.async_copy(rows_b.at[pl.ds(0, CH)],
                                       ussh.at[idx_b.at[0]], scb, add=True)
                sb2 = pltpu.async_copy(rows_b.at[pl.ds(CH, CH)],
                                       ussh.at[idx_b.at[1]], scb, add=True)
                sa1.wait()
                sa2.wait()
                sb1.wait()
                sb2.wait()

            @pl.when(jnp.logical_not(pid_b < NPAIRT))
            def _a_only():
                i_a.wait()
                l_a.wait()
                sa1 = pltpu.async_copy(rows_a.at[pl.ds(0, CH)],
                                       ussh.at[idx_a.at[0]], sca, add=True)
                sa2 = pltpu.async_copy(rows_a.at[pl.ds(CH, CH)],
                                       ussh.at[idx_a.at[1]], sca, add=True)
                sa1.wait()
                sa2.wait()

        return carry

    lax.fori_loop(0, NPITER // 2, body, 0)

    plsc.subcore_barrier()
    pltpu.sync_copy(ussh.at[pl.ds(tid * STRIPE, STRIPE)],
                    usum_hbm.at[pl.ds(c * N_PAD + tid * STRIPE, STRIPE)])


def _kdot_body(agg_ref, usum_ref, w0, b0, w1, b1, w2, b2, out_ref):
    a = agg_ref[0] + agg_ref[1]
    us = usum_ref[0] + usum_ref[1]
    h = jnp.tanh(jnp.dot(a, w0[...],
                         preferred_element_type=jnp.float32) + b0[...])
    h = jnp.maximum(jnp.dot(h, w1[...],
                            preferred_element_type=jnp.float32) + b1[...], 0.0)
    k = jnp.dot(h, w2[...], preferred_element_type=jnp.float32) + b2[...]
    part = jnp.sum(k * us).reshape(1, 1)

    @pl.when(pl.program_id(0) == 0)
    def _():
        out_ref[...] = jnp.zeros((1, 1), jnp.float32)

    out_ref[...] += part


def _kdot(agg, usum, w0, b0, w1, b1, w2, b2):
    rows = 2048
    grid = (N_PAD // rows,)
    wspec = pl.BlockSpec((D, D), lambda i: (0, 0))
    bspec = pl.BlockSpec((1, D), lambda i: (0, 0))
    out = pl.pallas_call(
        _kdot_body,
        grid=grid,
        in_specs=[pl.BlockSpec((NC, rows, D), lambda i: (0, i, 0)),
                  pl.BlockSpec((NC, rows, D), lambda i: (0, i, 0)),
                  wspec, bspec, wspec, bspec, wspec, bspec],
        out_specs=pl.BlockSpec((1, 1), lambda i: (0, 0)),
        out_shape=jax.ShapeDtypeStruct((1, 1), jnp.float32),
    )(agg, usum, w0, b0, w1, b1, w2, b2)
    return out[0, 0]


def kernel(x, edge_index, e,
           Wk0, bk0, Wk1, bk1, Wk2, bk2,
           Wu0, bu0, Wu1, bu1, Wu2, bu2,
           WencK, bencK, WencP1, bencP1, WencP2, bencP2):
    src = edge_index[0].reshape(NCHT, CH)
    dst = edge_index[1].reshape(NCHT, CH)
    src_a = edge_index[0].reshape(NCHTA, CHA)
    dst_a = edge_index[1].reshape(NCHTA, CHA)

    w_enc = jnp.concatenate([WencK.T, WencP1.T, WencP2.T], axis=1)
    b_enc = jnp.concatenate([bencK, bencP1, bencP2])[None, :]
    h1, h21, h22 = _encode(x, w_enc, b_enc)

    z = jnp.zeros((STRIPE, D), jnp.float32)
    s = _sc1a(src_a, dst_a, h21, h22)
    agg = _sc1b(src_a, dst_a, h1, z)

    u = _umlp(s, Wu0.T, bu0[None, :], Wu1.T, bu1[None, :], Wu2.T, bu2[None, :])

    usum = _sc2(src, u, z)

    agg3 = agg.reshape(NC, N_PAD, D)
    usum3 = usum.reshape(NC, N_PAD, D)
    return _kdot(agg3, usum3,
                 Wk0.T, bk0[None, :], Wk1.T, bk1[None, :], Wk2.T, bk2[None, :])


# speedup vs baseline: 1.2952x; 1.0375x over previous
import functools

import jax
import jax.numpy as jnp
from jax import lax
from jax.experimental import pallas as pl
from jax.experimental.pallas import tpu as pltpu
from jax.experimental.pallas import tpu_sc as plsc

N_NODES = 10000
N_EDGES = 320000
D = 128

NC = 2
NS = 16
NW = NC * NS
CH = 64
NCHT = N_EDGES // CH
NPAIRT = NCHT // 2
NPITER = 80
HALF_E = N_EDGES // 2
NPAIRH = NPAIRT // 2
NPITERH = 40
CHA = 128
NCHTA = N_EDGES // CHA
NPAIRA = NCHTA // 2
NPITERA = 40
N_PAD = 10240
STRIPE = N_PAD // NS

_MESH = plsc.VectorSubcoreMesh(core_axis_name="c", subcore_axis_name="s")


def _enc_body(x_ref, w_ref, b_ref, h1_ref, h21_ref, h22_ref):
    h = jnp.dot(x_ref[...], w_ref[...], preferred_element_type=jnp.float32)
    h = h + b_ref[...]
    h1_ref[...] = h[:, :D]
    h21_ref[...] = h[:, D:2 * D]
    h22_ref[...] = h[:, 2 * D:]


def _encode(x, w_enc, b_enc):
    rows = 2000
    grid = (N_NODES // rows,)
    return pl.pallas_call(
        _enc_body,
        grid=grid,
        in_specs=[
            pl.BlockSpec((rows, D), lambda i: (i, 0)),
            pl.BlockSpec((D, 3 * D), lambda i: (0, 0)),
            pl.BlockSpec((1, 3 * D), lambda i: (0, 0)),
        ],
        out_specs=[
            pl.BlockSpec((rows, D), lambda i: (i, 0)),
            pl.BlockSpec((rows, D), lambda i: (i, 0)),
            pl.BlockSpec((rows, D), lambda i: (i, 0)),
        ],
        out_shape=[jax.ShapeDtypeStruct((N_NODES, D), jnp.float32)] * 3,
    )(x, w_enc, b_enc)


@functools.partial(
    pl.kernel,
    out_type=jax.ShapeDtypeStruct((N_EDGES, D), jnp.float32),
    mesh=_MESH,
    scratch_types=[
        pltpu.VMEM((2, CHA), jnp.int32),
        pltpu.VMEM((2, CHA), jnp.int32),
        pltpu.VMEM((2, CHA), jnp.int32),
        pltpu.VMEM((2, CHA), jnp.int32),
        pltpu.VMEM((CHA, D), jnp.float32),
        pltpu.VMEM((CHA, D), jnp.float32),
        pltpu.VMEM((CHA, D), jnp.float32),
        pltpu.VMEM((CHA, D), jnp.float32),
    ] + [pltpu.SemaphoreType.DMA] * 12,
)
def _sc1a(src_hbm, dst_hbm, h21_hbm, h22_hbm, s_hbm,
          ixs0, ixd0, ixs1, ixd1, ra, rb, rc, rd,
          i0, i1, i2, i3, ga, gb, gc, gd, wsa, wsb, wsc, wsd):
    c = lax.axis_index("c")
    sidx = lax.axis_index("s")
    wid = sidx * NC + c

    def body(m, carry):
        p0 = (2 * m) * NW + wid
        p1 = (2 * m + 1) * NW + wid

        @pl.when(p0 < NPAIRA)
        def _p0():
            ia = pltpu.async_copy(src_hbm.at[pl.ds(2 * p0, 2)], ixs0, i0)
            ib = pltpu.async_copy(dst_hbm.at[pl.ds(2 * p0, 2)], ixd0, i1)

            @pl.when(p1 < NPAIRA)
            def _pre1():
                pltpu.async_copy(src_hbm.at[pl.ds(2 * p1, 2)], ixs1, i2)
                pltpu.async_copy(dst_hbm.at[pl.ds(2 * p1, 2)], ixd1, i3)

            ia.wait()
            g_a = pltpu.async_copy(h21_hbm.at[ixs0.at[0]], ra, ga)
            g_b = pltpu.async_copy(h21_hbm.at[ixs0.at[1]], rb, gb)

            @pl.when(p1 < NPAIRA)
            def _g1():
                pltpu.make_async_copy(src_hbm.at[pl.ds(2 * p1, 2)], ixs1,
                                      i2).wait()
                pltpu.async_copy(h21_hbm.at[ixs1.at[0]], rc, gc)
                pltpu.async_copy(h21_hbm.at[ixs1.at[1]], rd, gd)

            ib.wait()
            g_a.wait()
            a3 = pltpu.async_copy(h22_hbm.at[ixd0.at[0]], ra, ga, add=True)
            g_b.wait()
            b3 = pltpu.async_copy(h22_hbm.at[ixd0.at[1]], rb, gb, add=True)

            @pl.when(p1 < NPAIRA)
            def _g2():
                pltpu.make_async_copy(dst_hbm.at[pl.ds(2 * p1, 2)], ixd1,
                                      i3).wait()
                pltpu.make_async_copy(h21_hbm.at[ixs1.at[0]], rc, gc).wait()
                pltpu.async_copy(h22_hbm.at[ixd1.at[0]], rc, gc, add=True)
                pltpu.make_async_copy(h21_hbm.at[ixs1.at[1]], rd, gd).wait()
                pltpu.async_copy(h22_hbm.at[ixd1.at[1]], rd, gd, add=True)

            a3.wait()
            w_a = pltpu.async_copy(ra, s_hbm.at[pl.ds(2 * p0 * CHA, CHA)],
                                   wsa)
            b3.wait()
            w_b = pltpu.async_copy(rb,
                                   s_hbm.at[pl.ds((2 * p0 + 1) * CHA, CHA)],
                                   wsb)

            @pl.when(p1 < NPAIRA)
            def _w1():
                pltpu.make_async_copy(h22_hbm.at[ixd1.at[0]], rc, gc).wait()
                pltpu.async_copy(rc, s_hbm.at[pl.ds(2 * p1 * CHA, CHA)], wsc)
                pltpu.make_async_copy(h22_hbm.at[ixd1.at[1]], rd, gd).wait()
                pltpu.async_copy(rd,
                                 s_hbm.at[pl.ds((2 * p1 + 1) * CHA, CHA)],
                                 wsd)

            w_a.wait()
            w_b.wait()

            @pl.when(p1 < NPAIRA)
            def _dr1():
                pltpu.make_async_copy(
                    rc, s_hbm.at[pl.ds(2 * p1 * CHA, CHA)], wsc).wait()
                pltpu.make_async_copy(
                    rd, s_hbm.at[pl.ds((2 * p1 + 1) * CHA, CHA)], wsd).wait()

        return carry

    lax.fori_loop(0, NPITERA // 2, body, 0)


@functools.partial(
    pl.kernel,
    out_type=jax.ShapeDtypeStruct((NC * N_PAD, D), jnp.float32),
    mesh=_MESH,
    scratch_types=[
        pltpu.VMEM((2, CHA), jnp.int32),
        pltpu.VMEM((2, CHA), jnp.int32),
        pltpu.VMEM((CHA, D), jnp.float32),
        pltpu.VMEM((CHA, D), jnp.float32),
        pltpu.VMEM_SHARED((N_PAD, D), jnp.float32),
    ] + [pltpu.SemaphoreType.DMA] * 6,
)
def _sc1b(src_hbm, dst_hbm, h1_hbm, z_hbm,
          agg_hbm, idx_s, idx_d, rows_a, rows_b, aggsh,
          si1, si2, s1, s3, s5, s6):
    c = lax.axis_index("c")
    sidx = lax.axis_index("s")
    wid = sidx * NC + c
    tid = sidx

    pltpu.sync_copy(z_hbm, aggsh.at[pl.ds(tid * STRIPE, STRIPE)])
    plsc.subcore_barrier()

    def body(j, carry):
        pid = j * NW + wid

        @pl.when(pid < NPAIRA)
        def _pair():
            i1 = pltpu.async_copy(src_hbm.at[pl.ds(2 * pid, 2)], idx_s, si1)
            i2 = pltpu.async_copy(dst_hbm.at[pl.ds(2 * pid, 2)], idx_d, si2)
            i1.wait()
            g1a = pltpu.async_copy(h1_hbm.at[idx_s.at[0]], rows_a, s1)
            g1b = pltpu.async_copy(h1_hbm.at[idx_s.at[1]], rows_b, s3)
            i2.wait()
            g1a.wait()
            sca = pltpu.async_copy(rows_a, aggsh.at[idx_d.at[0]], s5,
                                   add=True)
            g1b.wait()
            scb = pltpu.async_copy(rows_b, aggsh.at[idx_d.at[1]], s6,
                                   add=True)
            sca.wait()
            scb.wait()

        return carry

    lax.fori_loop(0, NPITERA, body, 0)

    plsc.subcore_barrier()
    pltpu.sync_copy(aggsh.at[pl.ds(tid * STRIPE, STRIPE)],
                    agg_hbm.at[pl.ds(c * N_PAD + tid * STRIPE, STRIPE)])


def _umlp_body(s_ref, w0, b0, w1, b1, w2, b2, u_ref):
    h = jnp.tanh(jnp.dot(s_ref[...], w0[...],
                         preferred_element_type=jnp.float32) + b0[...])
    h = jnp.maximum(jnp.dot(h, w1[...],
                            preferred_element_type=jnp.float32) + b1[...], 0.0)
    u_ref[...] = jnp.dot(h, w2[...],
                         preferred_element_type=jnp.float32) + b2[...]


def _umlp(s, w0, b0, w1, b1, w2, b2):
    rows = 2000
    grid = (N_EDGES // rows,)
    wspec = pl.BlockSpec((D, D), lambda i: (0, 0))
    bspec = pl.BlockSpec((1, D), lambda i: (0, 0))
    return pl.pallas_call(
        _umlp_body,
        grid=grid,
        in_specs=[pl.BlockSpec((rows, D), lambda i: (i, 0)),
                  wspec, bspec, wspec, bspec, wspec, bspec],
        out_specs=pl.BlockSpec((rows, D), lambda i: (i, 0)),
        out_shape=jax.ShapeDtypeStruct((N_EDGES, D), jnp.float32),
    )(s, w0, b0, w1, b1, w2, b2)


@functools.partial(
    pl.kernel,
    out_type=jax.ShapeDtypeStruct((NC * N_PAD, D), jnp.float32),
    mesh=_MESH,
    scratch_types=[
        pltpu.VMEM((2, CH), jnp.int32),
        pltpu.VMEM((2, CH), jnp.int32),
        pltpu.VMEM((2 * CH, D), jnp.float32),
        pltpu.VMEM((2 * CH, D), jnp.float32),
        pltpu.VMEM_SHARED((N_PAD, D), jnp.float32),
    ] + [pltpu.SemaphoreType.DMA] * 6,
)
def _sc2(src_hbm, u_hbm, z_hbm, usum_hbm, idx_a, idx_b, rows_a, rows_b,
         ussh, si1, si2, sl1, sl2, sca, scb):
    c = lax.axis_index("c")
    sidx = lax.axis_index("s")
    wid = sidx * NC + c
    tid = sidx

    pltpu.sync_copy(z_hbm, ussh.at[pl.ds(tid * STRIPE, STRIPE)])
    plsc.subcore_barrier()

    def drain_b():
        pltpu.make_async_copy(rows_b.at[pl.ds(0, CH)],
                              ussh.at[idx_b.at[0]], scb).wait()
        pltpu.make_async_copy(rows_b.at[pl.ds(CH, CH)],
                              ussh.at[idx_b.at[1]], scb).wait()

    def body(m, carry):
        pid_a = (2 * m) * NW + wid
        pid_b = (2 * m + 1) * NW + wid
        prev_b = (2 * m - 1) * NW + wid

        @pl.when(pid_a < NPAIRT)
        def _a():
            i_a = pltpu.async_copy(src_hbm.at[pl.ds(2 * pid_a, 2)], idx_a,
                                   si1)
            l_a = pltpu.async_copy(u_hbm.at[pl.ds(2 * pid_a * CH, 2 * CH)],
                                   rows_a, sl1)
            i_a.wait()
            l_a.wait()
            sa1 = pltpu.async_copy(rows_a.at[pl.ds(0, CH)],
                                   ussh.at[idx_a.at[0]], sca, add=True)
            sa2 = pltpu.async_copy(rows_a.at[pl.ds(CH, CH)],
                                   ussh.at[idx_a.at[1]], sca, add=True)

            @pl.when(pid_b < NPAIRT)
            def _b():
                @pl.when(jnp.logical_and(m > 0, prev_b < NPAIRT))
                def _dprev():
                    drain_b()

                i_b = pltpu.async_copy(src_hbm.at[pl.ds(2 * pid_b, 2)],
                                       idx_b, si2)
                l_b = pltpu.async_copy(u_hbm.at[pl.ds(2 * pid_b * CH, 2 * CH)],
                                       rows_b, sl2)
                sa1.wait()
                sa2.wait()
                i_b.wait()
                l_b.wait()
                pltpu.async_copy(rows_b.at[pl.ds(0, CH)],
                                 ussh.at[idx_b.at[0]], scb, add=True)
                pltpu.async_copy(rows_b.at[pl.ds(CH, CH)],
                                 ussh.at[idx_b.at[1]], scb, add=True)

            @pl.when(jnp.logical_not(pid_b < NPAIRT))
            def _a_only():
                @pl.when(jnp.logical_and(m > 0, prev_b < NPAIRT))
                def _dprev2():
                    drain_b()

                sa1.wait()
                sa2.wait()

        return carry

    nouter = NPITER // 2
    lax.fori_loop(0, nouter, body, 0)

    last_a = (2 * nouter - 2) * NW + wid
    last_b = (2 * nouter - 1) * NW + wid
    prev_b = (2 * nouter - 3) * NW + wid

    @pl.when(jnp.logical_or(
        last_b < NPAIRT,
        jnp.logical_and(last_a >= NPAIRT, prev_b < NPAIRT)))
    def _dlast():
        drain_b()

    plsc.subcore_barrier()
    pltpu.sync_copy(ussh.at[pl.ds(tid * STRIPE, STRIPE)],
                    usum_hbm.at[pl.ds(c * N_PAD + tid * STRIPE, STRIPE)])


def _kdot_body(agg_ref, usum_ref, w0, b0, w1, b1, w2, b2, out_ref):
    a = agg_ref[0] + agg_ref[1]
    us = usum_ref[0] + usum_ref[1]
    h = jnp.tanh(jnp.dot(a, w0[...],
                         preferred_element_type=jnp.float32) + b0[...])
    h = jnp.maximum(jnp.dot(h, w1[...],
                            preferred_element_type=jnp.float32) + b1[...], 0.0)
    k = jnp.dot(h, w2[...], preferred_element_type=jnp.float32) + b2[...]
    part = jnp.sum(k * us).reshape(1, 1)

    @pl.when(pl.program_id(0) == 0)
    def _():
        out_ref[...] = jnp.zeros((1, 1), jnp.float32)

    out_ref[...] += part


def _kdot(agg, usum, w0, b0, w1, b1, w2, b2):
    rows = 2048
    grid = (N_PAD // rows,)
    wspec = pl.BlockSpec((D, D), lambda i: (0, 0))
    bspec = pl.BlockSpec((1, D), lambda i: (0, 0))
    out = pl.pallas_call(
        _kdot_body,
        grid=grid,
        in_specs=[pl.BlockSpec((NC, rows, D), lambda i: (0, i, 0)),
                  pl.BlockSpec((NC, rows, D), lambda i: (0, i, 0)),
                  wspec, bspec, wspec, bspec, wspec, bspec],
        out_specs=pl.BlockSpec((1, 1), lambda i: (0, 0)),
        out_shape=jax.ShapeDtypeStruct((1, 1), jnp.float32),
    )(agg, usum, w0, b0, w1, b1, w2, b2)
    return out[0, 0]


def kernel(x, edge_index, e,
           Wk0, bk0, Wk1, bk1, Wk2, bk2,
           Wu0, bu0, Wu1, bu1, Wu2, bu2,
           WencK, bencK, WencP1, bencP1, WencP2, bencP2):
    src = edge_index[0].reshape(NCHT, CH)
    dst = edge_index[1].reshape(NCHT, CH)
    src_a = edge_index[0].reshape(NCHTA, CHA)
    dst_a = edge_index[1].reshape(NCHTA, CHA)

    w_enc = jnp.concatenate([WencK.T, WencP1.T, WencP2.T], axis=1)
    b_enc = jnp.concatenate([bencK, bencP1, bencP2])[None, :]
    h1, h21, h22 = _encode(x, w_enc, b_enc)

    z = jnp.zeros((STRIPE, D), jnp.float32)
    s = _sc1a(src_a, dst_a, h21, h22)
    agg = _sc1b(src_a, dst_a, h1, z)

    u = _umlp(s, Wu0.T, bu0[None, :], Wu1.T, bu1[None, :], Wu2.T, bu2[None, :])

    usum = _sc2(src, u, z)

    agg3 = agg.reshape(NC, N_PAD, D)
    usum3 = usum.reshape(NC, N_PAD, D)
    return _kdot(agg3, usum3,
                 Wk0.T, bk0[None, :], Wk1.T, bk1[None, :], Wk2.T, bk2[None, :])


# speedup vs baseline: 1.3300x; 1.0269x over previous
import functools

import jax
import jax.numpy as jnp
from jax import lax
from jax.experimental import pallas as pl
from jax.experimental.pallas import tpu as pltpu
from jax.experimental.pallas import tpu_sc as plsc

N_NODES = 10000
N_EDGES = 320000
D = 128

NC = 2
NS = 16
NW = NC * NS
CH = 64
NCHT = N_EDGES // CH
NPAIRT = NCHT // 2
NPITER = 80
HALF_E = N_EDGES // 2
NPAIRH = NPAIRT // 2
NPITERH = 40
CHA = 128
NCHTA = N_EDGES // CHA
NPAIRA = NCHTA // 2
NPITERA = 40
N_PAD = 10240
STRIPE = N_PAD // NS

_MESH = plsc.VectorSubcoreMesh(core_axis_name="c", subcore_axis_name="s")


def _enc_body(x_ref, w_ref, b_ref, h1_ref, h21_ref, h22_ref):
    h = jnp.dot(x_ref[...], w_ref[...], preferred_element_type=jnp.float32)
    h = h + b_ref[...]
    h1_ref[...] = h[:, :D]
    h21_ref[...] = h[:, D:2 * D]
    h22_ref[...] = h[:, 2 * D:]


def _encode(x, w_enc, b_enc):
    rows = 2000
    grid = (N_NODES // rows,)
    return pl.pallas_call(
        _enc_body,
        grid=grid,
        in_specs=[
            pl.BlockSpec((rows, D), lambda i: (i, 0)),
            pl.BlockSpec((D, 3 * D), lambda i: (0, 0)),
            pl.BlockSpec((1, 3 * D), lambda i: (0, 0)),
        ],
        out_specs=[
            pl.BlockSpec((rows, D), lambda i: (i, 0)),
            pl.BlockSpec((rows, D), lambda i: (i, 0)),
            pl.BlockSpec((rows, D), lambda i: (i, 0)),
        ],
        out_shape=[jax.ShapeDtypeStruct((N_NODES, D), jnp.float32)] * 3,
    )(x, w_enc, b_enc)


@functools.partial(
    pl.kernel,
    out_type=jax.ShapeDtypeStruct((N_EDGES, D), jnp.float32),
    mesh=_MESH,
    scratch_types=[
        pltpu.VMEM((2, CHA), jnp.int32),
        pltpu.VMEM((2, CHA), jnp.int32),
        pltpu.VMEM((2, CHA), jnp.int32),
        pltpu.VMEM((2, CHA), jnp.int32),
        pltpu.VMEM((CHA, D), jnp.float32),
        pltpu.VMEM((CHA, D), jnp.float32),
        pltpu.VMEM((CHA, D), jnp.float32),
        pltpu.VMEM((CHA, D), jnp.float32),
    ] + [pltpu.SemaphoreType.DMA] * 12,
)
def _sc1a(src_hbm, dst_hbm, h21_hbm, h22_hbm, s_hbm,
          ixs0, ixd0, ixs1, ixd1, ra, rb, rc, rd,
          i0, i1, i2, i3, ga, gb, gc, gd, wsa, wsb, wsc, wsd):
    c = lax.axis_index("c")
    sidx = lax.axis_index("s")
    wid = sidx * NC + c

    def body(m, carry):
        p0 = (2 * m) * NW + wid
        p1 = (2 * m + 1) * NW + wid

        @pl.when(p0 < NPAIRA)
        def _p0():
            ia = pltpu.async_copy(src_hbm.at[pl.ds(2 * p0, 2)], ixs0, i0)
            ib = pltpu.async_copy(dst_hbm.at[pl.ds(2 * p0, 2)], ixd0, i1)

            @pl.when(p1 < NPAIRA)
            def _pre1():
                pltpu.async_copy(src_hbm.at[pl.ds(2 * p1, 2)], ixs1, i2)
                pltpu.async_copy(dst_hbm.at[pl.ds(2 * p1, 2)], ixd1, i3)

            ia.wait()
            g_a = pltpu.async_copy(h21_hbm.at[ixs0.at[0]], ra, ga)
            g_b = pltpu.async_copy(h21_hbm.at[ixs0.at[1]], rb, gb)

            @pl.when(p1 < NPAIRA)
            def _g1():
                pltpu.make_async_copy(src_hbm.at[pl.ds(2 * p1, 2)], ixs1,
                                      i2).wait()
                pltpu.async_copy(h21_hbm.at[ixs1.at[0]], rc, gc)
                pltpu.async_copy(h21_hbm.at[ixs1.at[1]], rd, gd)

            ib.wait()
            g_a.wait()
            a3 = pltpu.async_copy(h22_hbm.at[ixd0.at[0]], ra, ga, add=True)
            g_b.wait()
            b3 = pltpu.async_copy(h22_hbm.at[ixd0.at[1]], rb, gb, add=True)

            @pl.when(p1 < NPAIRA)
            def _g2():
                pltpu.make_async_copy(dst_hbm.at[pl.ds(2 * p1, 2)], ixd1,
                                      i3).wait()
                pltpu.make_async_copy(h21_hbm.at[ixs1.at[0]], rc, gc).wait()
                pltpu.async_copy(h22_hbm.at[ixd1.at[0]], rc, gc, add=True)
                pltpu.make_async_copy(h21_hbm.at[ixs1.at[1]], rd, gd).wait()
                pltpu.async_copy(h22_hbm.at[ixd1.at[1]], rd, gd, add=True)

            a3.wait()
            w_a = pltpu.async_copy(ra, s_hbm.at[pl.ds(2 * p0 * CHA, CHA)],
                                   wsa)
            b3.wait()
            w_b = pltpu.async_copy(rb,
                                   s_hbm.at[pl.ds((2 * p0 + 1) * CHA, CHA)],
                                   wsb)

            @pl.when(p1 < NPAIRA)
            def _w1():
                pltpu.make_async_copy(h22_hbm.at[ixd1.at[0]], rc, gc).wait()
                pltpu.async_copy(rc, s_hbm.at[pl.ds(2 * p1 * CHA, CHA)], wsc)
                pltpu.make_async_copy(h22_hbm.at[ixd1.at[1]], rd, gd).wait()
                pltpu.async_copy(rd,
                                 s_hbm.at[pl.ds((2 * p1 + 1) * CHA, CHA)],
                                 wsd)

            w_a.wait()
            w_b.wait()

            @pl.when(p1 < NPAIRA)
            def _dr1():
                pltpu.make_async_copy(
                    rc, s_hbm.at[pl.ds(2 * p1 * CHA, CHA)], wsc).wait()
                pltpu.make_async_copy(
                    rd, s_hbm.at[pl.ds((2 * p1 + 1) * CHA, CHA)], wsd).wait()

        return carry

    lax.fori_loop(0, NPITERA // 2, body, 0)


@functools.partial(
    pl.kernel,
    out_type=jax.ShapeDtypeStruct((NC * N_PAD, D), jnp.float32),
    mesh=_MESH,
    scratch_types=[
        pltpu.VMEM((2, CHA), jnp.int32),
        pltpu.VMEM((2, CHA), jnp.int32),
        pltpu.VMEM((CHA, D), jnp.float32),
        pltpu.VMEM((CHA, D), jnp.float32),
        pltpu.VMEM_SHARED((N_PAD, D), jnp.float32),
    ] + [pltpu.SemaphoreType.DMA] * 6,
)
def _sc1b(src_hbm, dst_hbm, h1_hbm, z_hbm,
          agg_hbm, idx_s, idx_d, rows_a, rows_b, aggsh,
          si1, si2, s1, s3, s5, s6):
    c = lax.axis_index("c")
    sidx = lax.axis_index("s")
    wid = sidx * NC + c
    tid = sidx

    pltpu.sync_copy(z_hbm, aggsh.at[pl.ds(tid * STRIPE, STRIPE)])
    plsc.subcore_barrier()

    def body(j, carry):
        pid = j * NW + wid

        @pl.when(pid < NPAIRA)
        def _pair():
            i1 = pltpu.async_copy(src_hbm.at[pl.ds(2 * pid, 2)], idx_s, si1)
            i2 = pltpu.async_copy(dst_hbm.at[pl.ds(2 * pid, 2)], idx_d, si2)
            i1.wait()
            g1a = pltpu.async_copy(h1_hbm.at[idx_s.at[0]], rows_a, s1)
            g1b = pltpu.async_copy(h1_hbm.at[idx_s.at[1]], rows_b, s3)
            i2.wait()
            g1a.wait()
            sca = pltpu.async_copy(rows_a, aggsh.at[idx_d.at[0]], s5,
                                   add=True)
            g1b.wait()
            scb = pltpu.async_copy(rows_b, aggsh.at[idx_d.at[1]], s6,
                                   add=True)
            sca.wait()
            scb.wait()

        return carry

    lax.fori_loop(0, NPITERA, body, 0)

    plsc.subcore_barrier()
    pltpu.sync_copy(aggsh.at[pl.ds(tid * STRIPE, STRIPE)],
                    agg_hbm.at[pl.ds(c * N_PAD + tid * STRIPE, STRIPE)])


def _umlp_body(s_ref, w0, b0, w1, b1, w2, b2, u_ref):
    h = jnp.tanh(jnp.dot(s_ref[...], w0[...],
                         preferred_element_type=jnp.float32) + b0[...])
    h = jnp.maximum(jnp.dot(h, w1[...],
                            preferred_element_type=jnp.float32) + b1[...], 0.0)
    u_ref[...] = jnp.dot(h, w2[...],
                         preferred_element_type=jnp.float32) + b2[...]


def _umlp(s, w0, b0, w1, b1, w2, b2):
    rows = 3200
    grid = (N_EDGES // rows,)
    wspec = pl.BlockSpec((D, D), lambda i: (0, 0))
    bspec = pl.BlockSpec((1, D), lambda i: (0, 0))
    return pl.pallas_call(
        _umlp_body,
        grid=grid,
        in_specs=[pl.BlockSpec((rows, D), lambda i: (i, 0)),
                  wspec, bspec, wspec, bspec, wspec, bspec],
        out_specs=pl.BlockSpec((rows, D), lambda i: (i, 0)),
        out_shape=jax.ShapeDtypeStruct((N_EDGES, D), jnp.float32),
    )(s, w0, b0, w1, b1, w2, b2)


@functools.partial(
    pl.kernel,
    out_type=jax.ShapeDtypeStruct((NC * N_PAD, D), jnp.float32),
    mesh=_MESH,
    scratch_types=[
        pltpu.VMEM((2, CH), jnp.int32),
        pltpu.VMEM((2, CH), jnp.int32),
        pltpu.VMEM((2 * CH, D), jnp.float32),
        pltpu.VMEM((2 * CH, D), jnp.float32),
        pltpu.VMEM_SHARED((N_PAD, D), jnp.float32),
    ] + [pltpu.SemaphoreType.DMA] * 6,
)
def _sc2(src_hbm, u_hbm, z_hbm, usum_hbm, idx_a, idx_b, rows_a, rows_b,
         ussh, si1, si2, sl1, sl2, sca, scb):
    c = lax.axis_index("c")
    sidx = lax.axis_index("s")
    wid = sidx * NC + c
    tid = sidx

    pltpu.sync_copy(z_hbm, ussh.at[pl.ds(tid * STRIPE, STRIPE)])
    plsc.subcore_barrier()

    def drain_b():
        pltpu.make_async_copy(rows_b.at[pl.ds(0, CH)],
                              ussh.at[idx_b.at[0]], scb).wait()
        pltpu.make_async_copy(rows_b.at[pl.ds(CH, CH)],
                              ussh.at[idx_b.at[1]], scb).wait()

    def body(m, carry):
        pid_a = (2 * m) * NW + wid
        pid_b = (2 * m + 1) * NW + wid
        prev_b = (2 * m - 1) * NW + wid

        @pl.when(pid_a < NPAIRT)
        def _a():
            i_a = pltpu.async_copy(src_hbm.at[pl.ds(2 * pid_a, 2)], idx_a,
                                   si1)
            l_a = pltpu.async_copy(u_hbm.at[pl.ds(2 * pid_a * CH, 2 * CH)],
                                   rows_a, sl1)
            i_a.wait()
            l_a.wait()
            sa1 = pltpu.async_copy(rows_a.at[pl.ds(0, CH)],
                                   ussh.at[idx_a.at[0]], sca, add=True)
            sa2 = pltpu.async_copy(rows_a.at[pl.ds(CH, CH)],
                                   ussh.at[idx_a.at[1]], sca, add=True)

            @pl.when(pid_b < NPAIRT)
            def _b():
                @pl.when(jnp.logical_and(m > 0, prev_b < NPAIRT))
                def _dprev():
                    drain_b()

                i_b = pltpu.async_copy(src_hbm.at[pl.ds(2 * pid_b, 2)],
                                       idx_b, si2)
                l_b = pltpu.async_copy(u_hbm.at[pl.ds(2 * pid_b * CH, 2 * CH)],
                                       rows_b, sl2)
                sa1.wait()
                sa2.wait()
                i_b.wait()
                l_b.wait()
                pltpu.async_copy(rows_b.at[pl.ds(0, CH)],
                                 ussh.at[idx_b.at[0]], scb, add=True)
                pltpu.async_copy(rows_b.at[pl.ds(CH, CH)],
                                 ussh.at[idx_b.at[1]], scb, add=True)

            @pl.when(jnp.logical_not(pid_b < NPAIRT))
            def _a_only():
                @pl.when(jnp.logical_and(m > 0, prev_b < NPAIRT))
                def _dprev2():
                    drain_b()

                sa1.wait()
                sa2.wait()

        return carry

    nouter = NPITER // 2
    lax.fori_loop(0, nouter, body, 0)

    last_a = (2 * nouter - 2) * NW + wid
    last_b = (2 * nouter - 1) * NW + wid
    prev_b = (2 * nouter - 3) * NW + wid

    @pl.when(jnp.logical_or(
        last_b < NPAIRT,
        jnp.logical_and(last_a >= NPAIRT, prev_b < NPAIRT)))
    def _dlast():
        drain_b()

    plsc.subcore_barrier()
    pltpu.sync_copy(ussh.at[pl.ds(tid * STRIPE, STRIPE)],
                    usum_hbm.at[pl.ds(c * N_PAD + tid * STRIPE, STRIPE)])


def _kdot_body(agg_ref, usum_ref, w0, b0, w1, b1, w2, b2, out_ref):
    a = agg_ref[0] + agg_ref[1]
    us = usum_ref[0] + usum_ref[1]
    h = jnp.tanh(jnp.dot(a, w0[...],
                         preferred_element_type=jnp.float32) + b0[...])
    h = jnp.maximum(jnp.dot(h, w1[...],
                            preferred_element_type=jnp.float32) + b1[...], 0.0)
    k = jnp.dot(h, w2[...], preferred_element_type=jnp.float32) + b2[...]
    part = jnp.sum(k * us).reshape(1, 1)

    @pl.when(pl.program_id(0) == 0)
    def _():
        out_ref[...] = jnp.zeros((1, 1), jnp.float32)

    out_ref[...] += part


def _kdot(agg, usum, w0, b0, w1, b1, w2, b2):
    rows = 2048
    grid = (N_PAD // rows,)
    wspec = pl.BlockSpec((D, D), lambda i: (0, 0))
    bspec = pl.BlockSpec((1, D), lambda i: (0, 0))
    out = pl.pallas_call(
        _kdot_body,
        grid=grid,
        in_specs=[pl.BlockSpec((NC, rows, D), lambda i: (0, i, 0)),
                  pl.BlockSpec((NC, rows, D), lambda i: (0, i, 0)),
                  wspec, bspec, wspec, bspec, wspec, bspec],
        out_specs=pl.BlockSpec((1, 1), lambda i: (0, 0)),
        out_shape=jax.ShapeDtypeStruct((1, 1), jnp.float32),
    )(agg, usum, w0, b0, w1, b1, w2, b2)
    return out[0, 0]


def kernel(x, edge_index, e,
           Wk0, bk0, Wk1, bk1, Wk2, bk2,
           Wu0, bu0, Wu1, bu1, Wu2, bu2,
           WencK, bencK, WencP1, bencP1, WencP2, bencP2):
    src = edge_index[0].reshape(NCHT, CH)
    dst = edge_index[1].reshape(NCHT, CH)
    src_a = edge_index[0].reshape(NCHTA, CHA)
    dst_a = edge_index[1].reshape(NCHTA, CHA)

    w_enc = jnp.concatenate([WencK.T, WencP1.T, WencP2.T], axis=1)
    b_enc = jnp.concatenate([bencK, bencP1, bencP2])[None, :]
    h1, h21, h22 = _encode(x, w_enc, b_enc)

    z = jnp.zeros((STRIPE, D), jnp.float32)
    s = _sc1a(src_a, dst_a, h21, h22)
    agg = _sc1b(src_a, dst_a, h1, z)

    u = _umlp(s, Wu0.T, bu0[None, :], Wu1.T, bu1[None, :], Wu2.T, bu2[None, :])

    usum = _sc2(src, u, z)

    agg3 = agg.reshape(NC, N_PAD, D)
    usum3 = usum.reshape(NC, N_PAD, D)
    return _kdot(agg3, usum3,
                 Wk0.T, bk0[None, :], Wk1.T, bk1[None, :], Wk2.T, bk2[None, :])
